# Initial kernel scaffold; baseline (speedup 1.0000x reference)
import jax
import jax.numpy as jnp
from jax.experimental import pallas as pl


def kernel(input, token_embedding_table):
    raise NotImplementedError("write your pallas kernel here")


# speedup vs baseline: 1.0646x; 1.0646x over previous
import functools

import jax
import jax.numpy as jnp
from jax import lax
from jax.experimental import pallas as pl
from jax.experimental.pallas import tpu as pltpu
from jax.experimental.pallas import tpu_sc as plsc

_B = 1024 * 50
_D = 1000
_NC = 2
_NS = 16
_NW = _NC * _NS
_BPW = _B // _NW
_CHUNK = 40
_NCHUNK = _BPW // _CHUNK

_mesh = plsc.VectorSubcoreMesh(core_axis_name="c", subcore_axis_name="s")


@functools.partial(
    pl.kernel,
    out_type=jax.ShapeDtypeStruct((_B, _D), jnp.float32),
    mesh=_mesh,
    compiler_params=pltpu.CompilerParams(use_tc_tiling_on_sc=False),
    scratch_types=[
        pltpu.VMEM((_BPW,), jnp.int32),
        pltpu.VMEM((_CHUNK, _D), jnp.float32),
        pltpu.VMEM_SHARED((1000, _D), jnp.float32),
        pltpu.SemaphoreType.DMA,
    ],
)
def _gather(idx_hbm, table_hbm, out_hbm, idx_v, rows_v, table_s, sem):
    cid = lax.axis_index("c")
    sid = lax.axis_index("s")
    wid = sid * _NC + cid
    base = wid * _BPW

    @pl.when(sid == 0)
    def _():
        pltpu.sync_copy(table_hbm, table_s)

    pltpu.sync_copy(idx_hbm.at[pl.ds(base, _BPW)], idx_v)
    plsc.subcore_barrier()

    def body(k, carry):
        pltpu.async_copy(
            table_s.at[idx_v.at[pl.ds(k * _CHUNK, _CHUNK)]], rows_v, sem
        ).wait()
        pltpu.sync_copy(rows_v, out_hbm.at[pl.ds(base + k * _CHUNK, _CHUNK)])
        return carry

    lax.fori_loop(0, _NCHUNK, body, 0)


def kernel(input, token_embedding_table):
    idx = input.reshape(-1).astype(jnp.int32)
    out = _gather(idx, token_embedding_table)
    return out.reshape(input.shape + (token_embedding_table.shape[1],))


# speedup vs baseline: 1.1401x; 1.0710x over previous
import functools

import jax
import jax.numpy as jnp
from jax import lax
from jax.experimental import pallas as pl
from jax.experimental.pallas import tpu as pltpu
from jax.experimental.pallas import tpu_sc as plsc

_B = 1024 * 50
_D = 1000
_NC = 2
_NS = 16
_NW = _NC * _NS
_BPW = _B // _NW
_CHUNK = 32
_NCHUNK = _BPW // _CHUNK

_mesh = plsc.VectorSubcoreMesh(core_axis_name="c", subcore_axis_name="s")


@functools.partial(
    pl.kernel,
    out_type=jax.ShapeDtypeStruct((_B, _D), jnp.float32),
    mesh=_mesh,
    compiler_params=pltpu.CompilerParams(use_tc_tiling_on_sc=False),
    scratch_types=[
        pltpu.VMEM((_BPW,), jnp.int32),
        pltpu.VMEM((_CHUNK, _D), jnp.float32),
        pltpu.VMEM((_CHUNK, _D), jnp.float32),
        pltpu.VMEM_SHARED((1000, _D), jnp.float32),
        pltpu.SemaphoreType.DMA,
        pltpu.SemaphoreType.DMA,
    ],
)
def _gather(idx_hbm, table_hbm, out_hbm, idx_v, rows_a, rows_b, table_s, gsem, ssem):
    cid = lax.axis_index("c")
    sid = lax.axis_index("s")
    wid = sid * _NC + cid
    base = wid * _BPW

    @pl.when(sid == 0)
    def _():
        pltpu.sync_copy(table_hbm, table_s)

    pltpu.sync_copy(idx_hbm.at[pl.ds(base, _BPW)], idx_v)
    plsc.subcore_barrier()

    bufs = (rows_a, rows_b)

    def start_gather(k, slot):
        pltpu.async_copy(
            table_s.at[idx_v.at[pl.ds(k * _CHUNK, _CHUNK)]], bufs[slot], gsem
        )

    def wait_gather(k, slot):
        pltpu.make_async_copy(
            table_s.at[idx_v.at[pl.ds(k * _CHUNK, _CHUNK)]], bufs[slot], gsem
        ).wait()

    def start_store(k, slot):
        pltpu.async_copy(
            bufs[slot], out_hbm.at[pl.ds(base + k * _CHUNK, _CHUNK)], ssem
        )

    def wait_store():
        pltpu.make_async_copy(
            rows_a, out_hbm.at[pl.ds(base, _CHUNK)], ssem
        ).wait()

    start_gather(0, 0)
    npair = _NCHUNK // 2

    def body(j, carry):
        k = 2 * j
        wait_gather(k, 0)

        @pl.when(j >= 1)
        def _():
            wait_store()

        start_store(k, 0)
        start_gather(k + 1, 1)
        wait_gather(k + 1, 1)
        wait_store()
        start_store(k + 1, 1)

        @pl.when(j + 1 < npair)
        def _():
            start_gather(k + 2, 0)

        return carry

    lax.fori_loop(0, npair, body, 0)
    wait_store()


def kernel(input, token_embedding_table):
    idx = input.reshape(-1).astype(jnp.int32)
    out = _gather(idx, token_embedding_table)
    return out.reshape(input.shape + (token_embedding_table.shape[1],))


# speedup vs baseline: 1.9349x; 1.6971x over previous
import functools

import jax
import jax.numpy as jnp
from jax import lax
from jax.experimental import pallas as pl
from jax.experimental.pallas import tpu as pltpu
from jax.experimental.pallas import tpu_sc as plsc

_BATCH = 1024
_T = 50
_TA = 48
_V = 1000
_D = 1000
_DP = 1024
_DA = 896
_NC = 2
_NS = 16
_NW = _NC * _NS
_BPW = _BATCH // _NW

_mesh = plsc.VectorSubcoreMesh(core_axis_name="c", subcore_axis_name="s")


@functools.partial(
    pl.kernel,
    out_type=(
        jax.ShapeDtypeStruct((_BATCH, _T, _D), jnp.float32),
        jax.ShapeDtypeStruct((_BATCH, _TA, _DP - _DA), jnp.float32),
        jax.ShapeDtypeStruct((_BATCH, _T - _TA, _DP), jnp.float32),
    ),
    mesh=_mesh,
    scratch_types=[
        pltpu.VMEM((_BPW, _TA), jnp.int32),
        pltpu.VMEM((_BPW, _T - _TA), jnp.int32),
        pltpu.VMEM((_TA, _DP), jnp.float32),
        pltpu.VMEM((_TA, _DP), jnp.float32),
        pltpu.VMEM((_T - _TA, _DP), jnp.float32),
        pltpu.VMEM((_T - _TA, _DP), jnp.float32),
        pltpu.SemaphoreType.DMA,
        pltpu.SemaphoreType.DMA,
    ],
)
def _gather(idxa_hbm, idxt_hbm, table_hbm, out_hbm, tail_hbm, trow_hbm,
            idxa_v, idxt_v, bufa0, bufa1, buft0, buft1, gsem, ssem):
    cid = lax.axis_index("c")
    sid = lax.axis_index("s")
    wid = sid * _NC + cid
    base = wid * _BPW
    pltpu.sync_copy(idxa_hbm.at[pl.ds(base, _BPW)], idxa_v)
    pltpu.sync_copy(idxt_hbm.at[pl.ds(base, _BPW)], idxt_v)

    bufsa = (bufa0, bufa1)
    bufst = (buft0, buft1)

    def start_gather(b, slot):
        pltpu.async_copy(table_hbm.at[idxa_v.at[b]], bufsa[slot], gsem)
        pltpu.async_copy(table_hbm.at[idxt_v.at[b]], bufst[slot], gsem)

    def wait_gather(b, slot):
        pltpu.make_async_copy(table_hbm.at[idxa_v.at[b]], bufsa[slot], gsem).wait()
        pltpu.make_async_copy(table_hbm.at[idxt_v.at[b]], bufst[slot], gsem).wait()

    def start_store(b, slot):
        pltpu.async_copy(
            bufsa[slot].at[:, pl.ds(0, _DA)],
            out_hbm.at[base + b, pl.ds(0, _TA), pl.ds(0, _DA)],
            ssem,
        )
        pltpu.async_copy(
            bufsa[slot].at[:, pl.ds(_DA, _DP - _DA)],
            tail_hbm.at[base + b],
            ssem,
        )
        pltpu.async_copy(bufst[slot], trow_hbm.at[base + b], ssem)

    def wait_store():
        pltpu.make_async_copy(
            bufa0.at[:, pl.ds(0, _DA)],
            out_hbm.at[base, pl.ds(0, _TA), pl.ds(0, _DA)],
            ssem,
        ).wait()
        pltpu.make_async_copy(
            bufa0.at[:, pl.ds(_DA, _DP - _DA)], tail_hbm.at[base], ssem
        ).wait()
        pltpu.make_async_copy(buft0, trow_hbm.at[base], ssem).wait()

    start_gather(0, 0)
    npair = _BPW // 2

    def body(j, carry):
        b = 2 * j
        wait_gather(b, 0)

        @pl.when(j >= 1)
        def _():
            wait_store()

        start_store(b, 0)
        start_gather(b + 1, 1)
        wait_gather(b + 1, 1)
        wait_store()
        start_store(b + 1, 1)

        @pl.when(j + 1 < npair)
        def _():
            start_gather(b + 2, 0)

        return carry

    lax.fori_loop(0, npair, body, 0)
    wait_store()


def kernel(input, token_embedding_table):
    idx = input.astype(jnp.int32)
    idxa = idx[:, :_TA]
    idxt = idx[:, _TA:]
    table_p = jnp.pad(token_embedding_table, ((0, 0), (0, _DP - _D)))
    main, tail, trow = _gather(idxa, idxt, table_p)
    out = lax.dynamic_update_slice(main, tail[:, :, : _D - _DA], (0, 0, _DA))
    out = lax.dynamic_update_slice(out, trow[:, :, :_D], (0, _TA, 0))
    return out
